# baseline (speedup 1.0000x reference)
import functools

import jax
import jax.numpy as jnp
from jax import lax
from jax.experimental import pallas as pl
from jax.experimental.pallas import tpu as pltpu
from jax.experimental.pallas import tpu_sc as plsc

B = 16384
DIM = 128
GAMMA = 12.0

NC = 2
NS = 16
L = 16
NW = NC * NS
BPW = B // NW
CH = 64
NCH = BPW // CH
NBUF = 3
UNR = 4
NGC = B // CH

_mesh = plsc.VectorSubcoreMesh(core_axis_name="c", subcore_axis_name="s")


@functools.partial(
    pl.kernel,
    mesh=_mesh,
    out_type=jax.ShapeDtypeStruct((B,), jnp.float32),
    scratch_types=[
        pltpu.VMEM((NCH, 2 * CH), jnp.int32),
        pltpu.VMEM((BPW,), jnp.int32),
    ] + [
        pltpu.VMEM((2 * CH, DIM), jnp.float32)
        for _ in range(NBUF)
    ] + [
        pltpu.VMEM((CH, DIM), jnp.float32)
        for _ in range(NBUF)
    ] + [
        pltpu.VMEM((BPW,), jnp.float32),
    ] + [pltpu.SemaphoreType.DMA for _ in range(NBUF)],
    compiler_params=pltpu.CompilerParams(needs_layout_passes=False),
)
def _sc_score(ht_hbm, ri_hbm, ent_hbm, rel_hbm, out_hbm,
              ht_v, ri_v, htb0, htb1, htb2, rb0, rb1, rb2,
              ob, sem0, sem1, sem2):
    cid = lax.axis_index("c")
    sid = lax.axis_index("s")
    wid = sid * NC + cid
    base = wid * BPW

    for hdl in (
        pltpu.async_copy(ht_hbm.at[pl.ds(wid * NCH, NCH)], ht_v, sem0),
        pltpu.async_copy(ri_hbm.at[pl.ds(base, BPW)], ri_v, sem0),
    ):
        hdl.wait()

    bufs = [(htb0, rb0, sem0), (htb1, rb1, sem1),
            (htb2, rb2, sem2)]

    def start(c):
        htb, rb, sem = bufs[c % NBUF]
        return (
            pltpu.async_copy(ent_hbm.at[ht_v.at[c]], htb, sem),
            pltpu.async_copy(rel_hbm.at[ri_v.at[pl.ds(c * CH, CH)]], rb, sem),
        )

    pending = [start(0), start(1)]
    for c in range(NCH):
        for hdl in pending.pop(0):
            hdl.wait()
        if c + 2 < NCH:
            pending.append(start(c + 2))
        htb, rb, _ = bufs[c % NBUF]
        for g in range(CH // L):
            lanes = lax.iota(jnp.int32, L)
            rows = lanes + (g * L)
            trows = rows + CH
            zero = jnp.zeros((L,), jnp.float32)

            @plsc.parallel_loop(0, DIM // UNR, unroll=2, carry=(zero,) * UNR)
            def accs(i, acc_in, htb=htb, rb=rb, rows=rows, trows=trows,
                     lanes=lanes):
                out = []
                for k in range(UNR):
                    cols = (lanes + (i * UNR + k)) & (DIM - 1)
                    h = plsc.load_gather(htb, [rows, cols])
                    r = plsc.load_gather(rb, [rows, cols])
                    t = plsc.load_gather(htb, [trows, cols])
                    out.append(acc_in[k] + jnp.abs(h + r - t))
                return tuple(out)

            acc = (accs[0] + accs[1]) + (accs[2] + accs[3])
            ob[pl.ds(c * CH + g * L, L)] = GAMMA - acc

    pltpu.sync_copy(ob, out_hbm.at[pl.ds(base, BPW)])


def kernel(sample, entity_embedding, relation_embedding):
    ht = jnp.concatenate(
        [sample[:, 0].reshape(NGC, CH), sample[:, 2].reshape(NGC, CH)],
        axis=1)
    ri = sample[:, 1]
    score = _sc_score(ht, ri, entity_embedding, relation_embedding)
    return score[:, None]


# speedup vs baseline: 1.0074x; 1.0074x over previous
import functools

import jax
import jax.numpy as jnp
from jax import lax
from jax.experimental import pallas as pl
from jax.experimental.pallas import tpu as pltpu
from jax.experimental.pallas import tpu_sc as plsc

B = 16384
DIM = 128
GAMMA = 12.0

NC = 2
NS = 16
L = 16
NW = NC * NS
BPW = B // NW
CH = 32
NCH = BPW // CH
NBUF = 4
UNR = 8
NGC = B // CH

_mesh = plsc.VectorSubcoreMesh(core_axis_name="c", subcore_axis_name="s")


@functools.partial(
    pl.kernel,
    mesh=_mesh,
    out_type=jax.ShapeDtypeStruct((B,), jnp.float32),
    scratch_types=[
        pltpu.VMEM((NCH, 2 * CH), jnp.int32),
        pltpu.VMEM((BPW,), jnp.int32),
    ] + [
        pltpu.VMEM((2 * CH, DIM), jnp.float32)
        for _ in range(NBUF)
    ] + [
        pltpu.VMEM((CH, DIM), jnp.float32)
        for _ in range(NBUF)
    ] + [
        pltpu.VMEM((BPW,), jnp.float32),
    ] + [pltpu.SemaphoreType.DMA for _ in range(NBUF)],
    compiler_params=pltpu.CompilerParams(needs_layout_passes=False),
)
def _sc_score(ht_hbm, ri_hbm, ent_hbm, rel_hbm, out_hbm,
              ht_v, ri_v, htb0, htb1, htb2, htb3, rb0, rb1, rb2, rb3,
              ob, sem0, sem1, sem2, sem3):
    cid = lax.axis_index("c")
    sid = lax.axis_index("s")
    wid = sid * NC + cid
    base = wid * BPW

    for hdl in (
        pltpu.async_copy(ht_hbm.at[pl.ds(wid * NCH, NCH)], ht_v, sem0),
        pltpu.async_copy(ri_hbm.at[pl.ds(base, BPW)], ri_v, sem0),
    ):
        hdl.wait()

    bufs = [(htb0, rb0, sem0), (htb1, rb1, sem1),
            (htb2, rb2, sem2), (htb3, rb3, sem3)]

    def start(c):
        htb, rb, sem = bufs[c % NBUF]
        return (
            pltpu.async_copy(ent_hbm.at[ht_v.at[c]], htb, sem),
            pltpu.async_copy(rel_hbm.at[ri_v.at[pl.ds(c * CH, CH)]], rb, sem),
        )

    pending = [start(0), start(1), start(2)]
    for c in range(NCH):
        for hdl in pending.pop(0):
            hdl.wait()
        if c + 3 < NCH:
            pending.append(start(c + 3))
        htb, rb, _ = bufs[c % NBUF]
        for g in range(CH // L):
            lanes = lax.iota(jnp.int32, L)
            rows = lanes + (g * L)
            trows = rows + CH
            zero = jnp.zeros((L,), jnp.float32)

            @plsc.parallel_loop(0, DIM // UNR, unroll=1, carry=(zero,) * UNR)
            def accs(i, acc_in, htb=htb, rb=rb, rows=rows, trows=trows,
                     lanes=lanes):
                out = []
                for k in range(UNR):
                    cols = (lanes + (i * UNR + k)) & (DIM - 1)
                    h = plsc.load_gather(htb, [rows, cols])
                    r = plsc.load_gather(rb, [rows, cols])
                    t = plsc.load_gather(htb, [trows, cols])
                    out.append(acc_in[k] + jnp.abs(h + r - t))
                return tuple(out)

            acc = ((accs[0] + accs[1]) + (accs[2] + accs[3])) + \
                  ((accs[4] + accs[5]) + (accs[6] + accs[7]))
            ob[pl.ds(c * CH + g * L, L)] = GAMMA - acc

    pltpu.sync_copy(ob, out_hbm.at[pl.ds(base, BPW)])


def kernel(sample, entity_embedding, relation_embedding):
    ht = jnp.concatenate(
        [sample[:, 0].reshape(NGC, CH), sample[:, 2].reshape(NGC, CH)],
        axis=1)
    ri = sample[:, 1]
    score = _sc_score(ht, ri, entity_embedding, relation_embedding)
    return score[:, None]


# speedup vs baseline: 1.0108x; 1.0034x over previous
import functools

import jax
import jax.numpy as jnp
from jax import lax
from jax.experimental import pallas as pl
from jax.experimental.pallas import tpu as pltpu
from jax.experimental.pallas import tpu_sc as plsc

B = 16384
DIM = 128
GAMMA = 12.0

NC = 2
NS = 16
L = 16
NW = NC * NS
BPW = B // NW
CH = 32
NCH = BPW // CH
NBUF = 4
UNR = 4
NGC = B // CH

_mesh = plsc.VectorSubcoreMesh(core_axis_name="c", subcore_axis_name="s")


@functools.partial(
    pl.kernel,
    mesh=_mesh,
    out_type=jax.ShapeDtypeStruct((B,), jnp.float32),
    scratch_types=[
        pltpu.VMEM((NCH, 2 * CH), jnp.int32),
        pltpu.VMEM((BPW,), jnp.int32),
    ] + [
        pltpu.VMEM((2 * CH, DIM), jnp.float32)
        for _ in range(NBUF)
    ] + [
        pltpu.VMEM((CH, DIM), jnp.float32)
        for _ in range(NBUF)
    ] + [
        pltpu.VMEM((BPW,), jnp.float32),
    ] + [pltpu.SemaphoreType.DMA for _ in range(NBUF)],
    compiler_params=pltpu.CompilerParams(needs_layout_passes=False),
)
def _sc_score(ht_hbm, ri_hbm, ent_hbm, rel_hbm, out_hbm,
              ht_v, ri_v, htb0, htb1, htb2, htb3, rb0, rb1, rb2, rb3,
              ob, sem0, sem1, sem2, sem3):
    cid = lax.axis_index("c")
    sid = lax.axis_index("s")
    wid = sid * NC + cid
    base = wid * BPW

    for hdl in (
        pltpu.async_copy(ht_hbm.at[pl.ds(wid * NCH, NCH)], ht_v, sem0),
        pltpu.async_copy(ri_hbm.at[pl.ds(base, BPW)], ri_v, sem0),
    ):
        hdl.wait()

    bufs = [(htb0, rb0, sem0), (htb1, rb1, sem1),
            (htb2, rb2, sem2), (htb3, rb3, sem3)]

    def start(c):
        htb, rb, sem = bufs[c % NBUF]
        return (
            pltpu.async_copy(ent_hbm.at[ht_v.at[c]], htb, sem),
            pltpu.async_copy(rel_hbm.at[ri_v.at[pl.ds(c * CH, CH)]], rb, sem),
        )

    pending = [start(0), start(1), start(2)]
    for c in range(NCH):
        for hdl in pending.pop(0):
            hdl.wait()
        if c + 3 < NCH:
            pending.append(start(c + 3))
        htb, rb, _ = bufs[c % NBUF]
        for g in range(CH // L):
            lanes = lax.iota(jnp.int32, L)
            rows = lanes + (g * L)
            trows = rows + CH
            zero = jnp.zeros((L,), jnp.float32)

            @plsc.parallel_loop(0, DIM // UNR, unroll=2, carry=(zero,) * UNR)
            def accs(i, acc_in, htb=htb, rb=rb, rows=rows, trows=trows,
                     lanes=lanes):
                out = []
                for k in range(UNR):
                    cols = (lanes + (i * UNR + k)) & (DIM - 1)
                    h = plsc.load_gather(htb, [rows, cols])
                    r = plsc.load_gather(rb, [rows, cols])
                    t = plsc.load_gather(htb, [trows, cols])
                    out.append(acc_in[k] + jnp.abs(h + r - t))
                return tuple(out)

            acc = (accs[0] + accs[1]) + (accs[2] + accs[3])
            ob[pl.ds(c * CH + g * L, L)] = GAMMA - acc

    pltpu.sync_copy(ob, out_hbm.at[pl.ds(base, BPW)])


def kernel(sample, entity_embedding, relation_embedding):
    ht = jnp.concatenate(
        [sample[:, 0].reshape(NGC, CH), sample[:, 2].reshape(NGC, CH)],
        axis=1)
    ri = sample[:, 1]
    score = _sc_score(ht, ri, entity_embedding, relation_embedding)
    return score[:, None]
